# baseline (speedup 1.0000x reference)
import functools

import jax
import jax.numpy as jnp
from jax import lax
from jax.experimental import pallas as pl
from jax.experimental.pallas import tpu as pltpu
from jax.experimental.pallas import tpu_sc as plsc

VOCAB = 100000
EMBED = 64
BATCH = 4096

NUM_CORES = 2
NUM_SUBCORES = 16
NUM_WORKERS = NUM_CORES * NUM_SUBCORES
B_PER_W = BATCH // NUM_WORKERS
LANES = 16


def _sc_body(iw_hbm, tw_hbm, we_hbm, wt_hbm, b_hbm, bt_hbm,
             dot_hbm, bsum_hbm,
             idx_i, idx_t, e_v, t_v, bi_v, bt_v, dot_v, bsum_v, sem):
    wid = lax.axis_index("s") * NUM_CORES + lax.axis_index("c")
    base = wid * B_PER_W

    pltpu.sync_copy(iw_hbm.at[pl.ds(base, B_PER_W)], idx_i)
    pltpu.sync_copy(tw_hbm.at[pl.ds(base, B_PER_W)], idx_t)

    c0 = pltpu.async_copy(we_hbm.at[idx_i], e_v, sem)
    c1 = pltpu.async_copy(wt_hbm.at[idx_t], t_v, sem)
    c2 = pltpu.async_copy(b_hbm.at[idx_i], bi_v, sem)
    c3 = pltpu.async_copy(bt_hbm.at[idx_t], bt_v, sem)
    c0.wait()
    c1.wait()
    c2.wait()
    c3.wait()

    lane = lax.iota(jnp.int32, LANES)
    for g in range(B_PER_W // LANES):
        row_idx = g * LANES + lane

        def col(c, acc):
            col_idx = jnp.full((LANES,), c, jnp.int32)
            ev = plsc.load_gather(e_v, [row_idx, col_idx])
            tv = plsc.load_gather(t_v, [row_idx, col_idx])
            return acc + ev * tv

        dot_v[pl.ds(g * LANES, LANES)] = lax.fori_loop(
            0, EMBED, col, jnp.zeros((LANES,), jnp.float32))

    for g in range(B_PER_W // LANES):
        s = pl.ds(g * LANES, LANES)
        bsum_v[s] = bi_v[s] + bt_v[s]

    pltpu.sync_copy(dot_v, dot_hbm.at[pl.ds(base, B_PER_W)])
    pltpu.sync_copy(bsum_v, bsum_hbm.at[pl.ds(base, B_PER_W)])


_sc_gather_dot = functools.partial(
    pl.kernel,
    out_type=(
        jax.ShapeDtypeStruct((BATCH,), jnp.float32),
        jax.ShapeDtypeStruct((BATCH,), jnp.float32),
    ),
    mesh=plsc.VectorSubcoreMesh(core_axis_name="c", subcore_axis_name="s"),
    compiler_params=pltpu.CompilerParams(
        needs_layout_passes=False, use_tc_tiling_on_sc=False),
    scratch_types=[
        pltpu.VMEM((B_PER_W,), jnp.int32),
        pltpu.VMEM((B_PER_W,), jnp.int32),
        pltpu.VMEM((B_PER_W, EMBED), jnp.float32),
        pltpu.VMEM((B_PER_W, EMBED), jnp.float32),
        pltpu.VMEM((B_PER_W,), jnp.float32),
        pltpu.VMEM((B_PER_W,), jnp.float32),
        pltpu.VMEM((B_PER_W,), jnp.float32),
        pltpu.VMEM((B_PER_W,), jnp.float32),
        pltpu.SemaphoreType.DMA,
    ],
)(_sc_body)


def _tc_body(bsum_ref, dot_ref, out_ref):
    out_ref[...] = bsum_ref[...] + dot_ref[...]


_BM = 256


@jax.jit
def _broadcast_add(bsum, dot):
    return pl.pallas_call(
        _tc_body,
        grid=(BATCH // _BM,),
        in_specs=[
            pl.BlockSpec((_BM, 1), lambda i: (i, 0)),
            pl.BlockSpec((1, BATCH), lambda i: (0, 0)),
        ],
        out_specs=pl.BlockSpec((_BM, BATCH), lambda i: (i, 0)),
        out_shape=jax.ShapeDtypeStruct((BATCH, BATCH), jnp.float32),
        compiler_params=pltpu.CompilerParams(
            dimension_semantics=("arbitrary",),
        ),
    )(bsum, dot)


@jax.jit
def kernel(input_word, target_word, W_embed, W_tilda, b, b_tilda):
    iw = input_word.astype(jnp.int32)
    tw = target_word.astype(jnp.int32)
    dot, bsum = _sc_gather_dot(iw, tw, W_embed, W_tilda,
                               b.reshape(-1), b_tilda.reshape(-1))
    return _broadcast_add(bsum.reshape(BATCH, 1), dot.reshape(1, BATCH))
